# baseline (speedup 1.0000x reference)
import functools

import jax
import jax.numpy as jnp
from jax import lax
from jax.experimental import pallas as pl
from jax.experimental.pallas import tpu as pltpu
from jax.experimental.pallas import tpu_sc as plsc

_BATCH = 128
_VOCAB = 100000
_K = 50
_KPAD = 64
_TOP_P = 0.9
_TEMP = 0.8

_NWORKERS = 32
_ROWS_PER_W = _BATCH // _NWORKERS
_NVREG = _VOCAB // 16
_CHUNK = 10
_NCHUNK = _NVREG // _CHUNK
_CAP = 512
_BIG = 2 ** 30


def _sc_body(logits, topv, topi, rowbuf, cand_v, cand_i, tmp_v, tmp_i,
             thrbuf, digbuf, selbuf, ptr_ref):
    iota16 = lax.iota(jnp.int32, 16)
    neg16 = jnp.full((16,), -jnp.inf, jnp.float32)
    big16 = jnp.full((16,), _BIG, jnp.int32)
    zero16 = jnp.zeros((16,), jnp.float32)
    zeroi16 = jnp.zeros((16,), jnp.int32)
    one16 = jnp.ones((16,), jnp.int32)
    step16 = jnp.full((16,), _CHUNK * 16, jnp.int32)
    thr_lane = jnp.full((16,), (_K - 1) % 16, jnp.int32)
    rot_idx = (iota16 + 15) & 15

    wid = lax.axis_index("c") * 16 + lax.axis_index("s")

    def _g(v, idx):
        return v.at[idx].get(mode="promise_in_bounds")

    def _bmax(v):
        for sh in (8, 4, 2, 1):
            v = jnp.maximum(v, _g(v, iota16 ^ sh))
        return v

    def _bmin(v):
        for sh in (8, 4, 2, 1):
            v = jnp.minimum(v, _g(v, iota16 ^ sh))
        return v

    def _bsum(v):
        for sh in (8, 4, 2, 1):
            v = v + _g(v, iota16 ^ sh)
        return v

    def extract_top():
        def zap(j, _):
            cand_v[pl.ds(j * 16, 16)] = neg16
            return 0
        lax.fori_loop(ptr_ref[0] // 16, _CAP // 16, zap, 0)

        d0, d1 = neg16, neg16
        for j in range(_CAP // 16):
            dv = _bmax(cand_v[pl.ds(j * 16, 16)])
            sel = iota16 == (j % 16)
            if j < 16:
                d0 = jnp.where(sel, dv, d0)
            else:
                d1 = jnp.where(sel, dv, d1)
        digbuf[pl.ds(0, 16)] = d0
        digbuf[pl.ds(16, 16)] = d1

        def ext(t, onehot):
            d0 = digbuf[pl.ds(0, 16)]
            d1 = digbuf[pl.ds(16, 16)]
            m16_ = _bmax(jnp.maximum(d0, d1))
            eq0 = d0 == m16_
            eq1 = d1 == m16_
            j16 = _bmin(jnp.minimum(jnp.where(eq0, iota16, big16),
                                    jnp.where(eq1, iota16 + 16, big16)))
            nt16 = _bsum(jnp.where(eq0, one16, zeroi16) +
                         jnp.where(eq1, one16, zeroi16))
            js = j16[0]

            @pl.when(nt16[0] == 1)
            def _():
                v = cand_v[pl.ds(js * 16, 16)]
                vi = cand_i[pl.ds(js * 16, 16)]
                ti16 = _bmin(jnp.where(v == m16_, vi, big16))
                selbuf[...] = ti16
                v2 = jnp.where(vi == ti16, neg16, v)
                cand_v[pl.ds(js * 16, 16)] = v2
                ndv = _bmax(v2)
                digbuf[pl.ds(0, 16)] = jnp.where(iota16 == j16, ndv, d0)
                digbuf[pl.ds(16, 16)] = jnp.where(iota16 + 16 == j16, ndv, d1)

            @pl.when(nt16[0] > 1)
            def _():
                def msweep(j, acc):
                    v = cand_v[pl.ds(j * 16, 16)]
                    vi = cand_i[pl.ds(j * 16, 16)]
                    return jnp.minimum(acc, jnp.where(v == m16_, vi, big16))
                ti16 = _bmin(lax.fori_loop(0, _CAP // 16, msweep, big16))
                selbuf[...] = ti16

                def killall(j, _):
                    vi = cand_i[pl.ds(j * 16, 16)]
                    cand_v[pl.ds(j * 16, 16)] = jnp.where(
                        vi == ti16, neg16, cand_v[pl.ds(j * 16, 16)])
                    return 0
                lax.fori_loop(0, _CAP // 16, killall, 0)
                nd0, nd1 = neg16, neg16
                for j in range(_CAP // 16):
                    dv = _bmax(cand_v[pl.ds(j * 16, 16)])
                    sel = iota16 == (j % 16)
                    if j < 16:
                        nd0 = jnp.where(sel, dv, nd0)
                    else:
                        nd1 = jnp.where(sel, dv, nd1)
                digbuf[pl.ds(0, 16)] = nd0
                digbuf[pl.ds(16, 16)] = nd1

            ti16 = selbuf[...]
            slot = t // 16 * 16
            lane_sel = onehot > zeroi16
            tmp_v[pl.ds(slot, 16)] = jnp.where(lane_sel, m16_,
                                               tmp_v[pl.ds(slot, 16)])
            tmp_i[pl.ds(slot, 16)] = jnp.where(lane_sel, ti16,
                                               tmp_i[pl.ds(slot, 16)])
            return _g(onehot, rot_idx)
        lax.fori_loop(0, _K, ext, jnp.where(iota16 == 0, one16, zeroi16))

    def row_body(r, _):
        row = wid * _ROWS_PER_W + r
        pltpu.sync_copy(logits.at[row], rowbuf)

        def zap_all(j, _):
            cand_v[pl.ds(j * 16, 16)] = neg16
            return 0
        lax.fori_loop(0, _CAP // 16, zap_all, 0)
        thrbuf[...] = neg16
        ptr_ref[0] = 0

        def chunk_body(ci_, idx16):
            base = ci_ * (_CHUNK * 16)
            thr16 = thrbuf[...]
            acc_or = None
            for k in range(_CHUNK):
                v = rowbuf[pl.ds(base + k * 16, 16)]
                mk = v > thr16
                acc_or = mk if acc_or is None else (acc_or | mk)
            any16 = _bmax(jnp.where(acc_or, one16, zeroi16))

            @pl.when(any16[0] > 0)
            def _():
                for k in range(_CHUNK):
                    kk16 = jnp.full((16,), k * 16, jnp.int32)
                    v = rowbuf[pl.ds(base + k * 16, 16)]
                    t16 = thrbuf[...]
                    mk = v > t16
                    a16 = _bmax(jnp.where(mk, one16, zeroi16))

                    @pl.when(a16[0] > 0)
                    def _(v=v, mk=mk, kk16=kk16):
                        p = ptr_ref[0]
                        cand_v[pl.ds(p, 16)] = jnp.where(mk, v, neg16)
                        cand_i[pl.ds(p, 16)] = idx16 + kk16
                        ptr_ref[0] = p + 16

                        @pl.when(p + 16 >= _CAP - 16)
                        def _():
                            extract_top()
                            for j in range(3):
                                cand_v[pl.ds(j * 16, 16)] = \
                                    tmp_v[pl.ds(j * 16, 16)]
                                cand_i[pl.ds(j * 16, 16)] = \
                                    tmp_i[pl.ds(j * 16, 16)]
                            cand_v[pl.ds(48, 16)] = jnp.where(
                                iota16 < 2, tmp_v[pl.ds(48, 16)], neg16)
                            cand_i[pl.ds(48, 16)] = tmp_i[pl.ds(48, 16)]

                            def zapr(j, _):
                                cand_v[pl.ds(j * 16, 16)] = neg16
                                return 0
                            lax.fori_loop(_KPAD // 16, _CAP // 16, zapr, 0)
                            thrbuf[...] = _g(tmp_v[pl.ds(48, 16)], thr_lane)
                            ptr_ref[0] = _KPAD

            return idx16 + step16

        lax.fori_loop(0, _NCHUNK, chunk_body, iota16)

        extract_top()
        pltpu.sync_copy(tmp_v, topv.at[row])
        pltpu.sync_copy(tmp_i, topi.at[row])
        return 0

    lax.fori_loop(0, _ROWS_PER_W, row_body, 0)


_sc_topk = functools.partial(
    pl.kernel,
    out_type=[
        jax.ShapeDtypeStruct((_BATCH, _KPAD), jnp.float32),
        jax.ShapeDtypeStruct((_BATCH, _KPAD), jnp.int32),
    ],
    mesh=plsc.VectorSubcoreMesh(core_axis_name="c", subcore_axis_name="s"),
    scratch_types=[
        pltpu.VMEM((_VOCAB,), jnp.float32),
        pltpu.VMEM((_CAP,), jnp.float32),
        pltpu.VMEM((_CAP,), jnp.int32),
        pltpu.VMEM((_KPAD,), jnp.float32),
        pltpu.VMEM((_KPAD,), jnp.int32),
        pltpu.VMEM((16,), jnp.float32),
        pltpu.VMEM((32,), jnp.float32),
        pltpu.VMEM((16,), jnp.int32),
        pltpu.SMEM((1,), jnp.int32),
    ],
)(_sc_body)


def _tail_body(topv_ref, topi_ref, stats_ref, g_ref, tok_ref, probs_ref):
    tv = topv_ref[...]
    ti = topi_ref[...]
    st = stats_ref[...]
    g = g_ref[...]
    m = st[:, 0:1]
    s = st[:, 1:2]
    col = lax.broadcasted_iota(jnp.int32, (_BATCH, _KPAD), 1)
    valid = col < _K
    vals = jnp.where(valid, jnp.exp(tv - m) / s, -jnp.inf)
    mx = jnp.max(vals, axis=-1, keepdims=True)
    e = jnp.where(valid, jnp.exp(vals - mx), 0.0)
    sm = e / jnp.sum(e, axis=-1, keepdims=True)
    c = sm
    for sh in (1, 2, 4, 8, 16, 32):
        c = c + jnp.concatenate(
            [jnp.zeros((_BATCH, sh), c.dtype), c[:, :-sh]], axis=-1)
    gt = c > _TOP_P
    any_gt = jnp.any(gt, axis=-1)
    first = jnp.where(any_gt,
                      jnp.min(jnp.where(gt, col, _KPAD), axis=-1), 0)
    first = jnp.minimum(first, _K)
    mask = col < first[:, None]
    num = jnp.exp(c / _TEMP) * mask.astype(jnp.float32)
    den = jnp.sum(num, axis=-1, keepdims=True)
    probs = num / den
    logp = jnp.where(mask, jnp.log(jnp.maximum(probs, 1e-30)), -jnp.inf)
    z = jnp.where(valid, logp + g, -jnp.inf)
    zm = jnp.max(z, axis=-1, keepdims=True)
    j = jnp.min(jnp.where(z == zm, col, _KPAD), axis=-1)
    tok_ref[...] = jnp.sum(jnp.where(col == j[:, None], ti, 0),
                           axis=-1, keepdims=True)
    probs_ref[...] = probs


_VPAD = 102400
_SBLK = _VPAD // 8


def _stats_body(x_ref, o_ref, m_ref, s_ref):
    i = pl.program_id(0)

    @pl.when(i == 0)
    def _():
        m_ref[...] = jnp.full((_BATCH, 1), -jnp.inf, jnp.float32)
        s_ref[...] = jnp.zeros((_BATCH, 1), jnp.float32)

    x = x_ref[...]
    bm = jnp.max(x, axis=-1, keepdims=True)
    m_old = m_ref[...]
    m_new = jnp.maximum(m_old, bm)
    s_ref[...] = (s_ref[...] * jnp.exp(m_old - m_new)
                  + jnp.sum(jnp.exp(x - m_new), axis=-1, keepdims=True))
    m_ref[...] = m_new

    @pl.when(i == pl.num_programs(0) - 1)
    def _():
        col = lax.broadcasted_iota(jnp.int32, (_BATCH, 16), 1)
        o_ref[...] = jnp.where(col == 0, m_ref[...],
                               jnp.where(col == 1, s_ref[...], 0.0))


def _tc_stats(padded):
    return pl.pallas_call(
        _stats_body,
        grid=(8,),
        in_specs=[pl.BlockSpec((_BATCH, _SBLK), lambda i: (0, i))],
        out_specs=pl.BlockSpec((_BATCH, 16), lambda i: (0, 0)),
        out_shape=jax.ShapeDtypeStruct((_BATCH, 16), jnp.float32),
        scratch_shapes=[
            pltpu.VMEM((_BATCH, 1), jnp.float32),
            pltpu.VMEM((_BATCH, 1), jnp.float32),
        ],
    )(padded)


def kernel(logits, top_k):
    del top_k
    padded = jnp.pad(logits, ((0, 0), (0, _VPAD - _VOCAB)),
                     constant_values=-jnp.inf)
    stats = _tc_stats(padded)
    topv, topi = _sc_topk(logits)
    g = jax.random.gumbel(jax.random.key(42), (_BATCH, _K), jnp.float32)
    g = jnp.pad(g, ((0, 0), (0, _KPAD - _K)))
    tok, probs = pl.pallas_call(
        _tail_body,
        out_shape=[
            jax.ShapeDtypeStruct((_BATCH, 1), jnp.int32),
            jax.ShapeDtypeStruct((_BATCH, _KPAD), jnp.float32),
        ],
    )(topv, topi, stats, g)
    return tok[:, 0], probs[:, :_K]


# speedup vs baseline: 1.3367x; 1.3367x over previous
import functools

import jax
import jax.numpy as jnp
from jax import lax
from jax.experimental import pallas as pl
from jax.experimental.pallas import tpu as pltpu
from jax.experimental.pallas import tpu_sc as plsc

_BATCH = 128
_VOCAB = 100000
_K = 50
_KPAD = 64
_TOP_P = 0.9
_TEMP = 0.8

_NWORKERS = 32
_ROWS_PER_W = _BATCH // _NWORKERS
_NVREG = _VOCAB // 16
_CHUNK = 10
_NCHUNK = _NVREG // _CHUNK
_CAP = 512
_BIG = 2 ** 30


def _sc_body(logits, topv, topi, stats, rowbuf, cand_v, cand_i, tmp_v, tmp_i,
             statbuf, thrbuf, digbuf, selbuf, ptr_ref):
    iota16 = lax.iota(jnp.int32, 16)
    neg16 = jnp.full((16,), -jnp.inf, jnp.float32)
    big16 = jnp.full((16,), _BIG, jnp.int32)
    zero16 = jnp.zeros((16,), jnp.float32)
    zeroi16 = jnp.zeros((16,), jnp.int32)
    one16 = jnp.ones((16,), jnp.int32)
    step16 = jnp.full((16,), _CHUNK * 16, jnp.int32)
    thr_lane = jnp.full((16,), (_K - 1) % 16, jnp.int32)
    rot_idx = (iota16 + 15) & 15

    wid = lax.axis_index("c") * 16 + lax.axis_index("s")

    def _g(v, idx):
        return v.at[idx].get(mode="promise_in_bounds")

    def _bmax(v):
        for sh in (8, 4, 2, 1):
            v = jnp.maximum(v, _g(v, iota16 ^ sh))
        return v

    def _bmin(v):
        for sh in (8, 4, 2, 1):
            v = jnp.minimum(v, _g(v, iota16 ^ sh))
        return v

    def _bsum(v):
        for sh in (8, 4, 2, 1):
            v = v + _g(v, iota16 ^ sh)
        return v

    def extract_top():
        def zap(j, _):
            cand_v[pl.ds(j * 16, 16)] = neg16
            return 0
        lax.fori_loop(ptr_ref[0] // 16, _CAP // 16, zap, 0)

        d0, d1 = neg16, neg16
        for j in range(_CAP // 16):
            dv = _bmax(cand_v[pl.ds(j * 16, 16)])
            sel = iota16 == (j % 16)
            if j < 16:
                d0 = jnp.where(sel, dv, d0)
            else:
                d1 = jnp.where(sel, dv, d1)
        digbuf[pl.ds(0, 16)] = d0
        digbuf[pl.ds(16, 16)] = d1

        def ext(t, onehot):
            d0 = digbuf[pl.ds(0, 16)]
            d1 = digbuf[pl.ds(16, 16)]
            m16_ = _bmax(jnp.maximum(d0, d1))
            eq0 = d0 == m16_
            eq1 = d1 == m16_
            j16 = _bmin(jnp.minimum(jnp.where(eq0, iota16, big16),
                                    jnp.where(eq1, iota16 + 16, big16)))
            nt16 = _bsum(jnp.where(eq0, one16, zeroi16) +
                         jnp.where(eq1, one16, zeroi16))
            js = j16[0]

            @pl.when(nt16[0] == 1)
            def _():
                v = cand_v[pl.ds(js * 16, 16)]
                vi = cand_i[pl.ds(js * 16, 16)]
                ti16 = _bmin(jnp.where(v == m16_, vi, big16))
                selbuf[...] = ti16
                v2 = jnp.where(vi == ti16, neg16, v)
                cand_v[pl.ds(js * 16, 16)] = v2
                ndv = _bmax(v2)
                digbuf[pl.ds(0, 16)] = jnp.where(iota16 == j16, ndv, d0)
                digbuf[pl.ds(16, 16)] = jnp.where(iota16 + 16 == j16, ndv, d1)

            @pl.when(nt16[0] > 1)
            def _():
                def msweep(j, acc):
                    v = cand_v[pl.ds(j * 16, 16)]
                    vi = cand_i[pl.ds(j * 16, 16)]
                    return jnp.minimum(acc, jnp.where(v == m16_, vi, big16))
                ti16 = _bmin(lax.fori_loop(0, _CAP // 16, msweep, big16))
                selbuf[...] = ti16

                def killall(j, _):
                    vi = cand_i[pl.ds(j * 16, 16)]
                    cand_v[pl.ds(j * 16, 16)] = jnp.where(
                        vi == ti16, neg16, cand_v[pl.ds(j * 16, 16)])
                    return 0
                lax.fori_loop(0, _CAP // 16, killall, 0)
                nd0, nd1 = neg16, neg16
                for j in range(_CAP // 16):
                    dv = _bmax(cand_v[pl.ds(j * 16, 16)])
                    sel = iota16 == (j % 16)
                    if j < 16:
                        nd0 = jnp.where(sel, dv, nd0)
                    else:
                        nd1 = jnp.where(sel, dv, nd1)
                digbuf[pl.ds(0, 16)] = nd0
                digbuf[pl.ds(16, 16)] = nd1

            ti16 = selbuf[...]
            slot = t // 16 * 16
            lane_sel = onehot > zeroi16
            tmp_v[pl.ds(slot, 16)] = jnp.where(lane_sel, m16_,
                                               tmp_v[pl.ds(slot, 16)])
            tmp_i[pl.ds(slot, 16)] = jnp.where(lane_sel, ti16,
                                               tmp_i[pl.ds(slot, 16)])
            return _g(onehot, rot_idx)
        lax.fori_loop(0, _K, ext, jnp.where(iota16 == 0, one16, zeroi16))

    def row_body(r, _):
        row = wid * _ROWS_PER_W + r
        pltpu.sync_copy(logits.at[row], rowbuf)

        def zap_all(j, _):
            cand_v[pl.ds(j * 16, 16)] = neg16
            return 0
        lax.fori_loop(0, _CAP // 16, zap_all, 0)
        thrbuf[...] = neg16
        ptr_ref[0] = 0

        def max_body(i, acc):
            return jnp.maximum(acc, rowbuf[pl.ds(i * 16, 16)])
        m16 = _bmax(lax.fori_loop(0, _NVREG, max_body, neg16, unroll=10))

        def chunk_body(ci_, carry):
            s_acc, idx16 = carry
            base = ci_ * (_CHUNK * 16)
            thr16 = thrbuf[...]
            acc_or = None
            for k in range(_CHUNK):
                v = rowbuf[pl.ds(base + k * 16, 16)]
                s_acc = s_acc + jnp.exp(v - m16)
                mk = v > thr16
                acc_or = mk if acc_or is None else (acc_or | mk)
            any16 = _bmax(jnp.where(acc_or, one16, zeroi16))

            @pl.when(any16[0] > 0)
            def _():
                for k in range(_CHUNK):
                    kk16 = jnp.full((16,), k * 16, jnp.int32)
                    v = rowbuf[pl.ds(base + k * 16, 16)]
                    t16 = thrbuf[...]
                    mk = v > t16
                    a16 = _bmax(jnp.where(mk, one16, zeroi16))

                    @pl.when(a16[0] > 0)
                    def _(v=v, mk=mk, kk16=kk16):
                        p = ptr_ref[0]
                        cand_v[pl.ds(p, 16)] = jnp.where(mk, v, neg16)
                        cand_i[pl.ds(p, 16)] = idx16 + kk16
                        ptr_ref[0] = p + 16

                        @pl.when(p + 16 >= _CAP - 16)
                        def _():
                            extract_top()
                            for j in range(3):
                                cand_v[pl.ds(j * 16, 16)] = \
                                    tmp_v[pl.ds(j * 16, 16)]
                                cand_i[pl.ds(j * 16, 16)] = \
                                    tmp_i[pl.ds(j * 16, 16)]
                            cand_v[pl.ds(48, 16)] = jnp.where(
                                iota16 < 2, tmp_v[pl.ds(48, 16)], neg16)
                            cand_i[pl.ds(48, 16)] = tmp_i[pl.ds(48, 16)]

                            def zapr(j, _):
                                cand_v[pl.ds(j * 16, 16)] = neg16
                                return 0
                            lax.fori_loop(_KPAD // 16, _CAP // 16, zapr, 0)
                            thrbuf[...] = _g(tmp_v[pl.ds(48, 16)], thr_lane)
                            ptr_ref[0] = _KPAD

            return s_acc, idx16 + step16

        s_acc, _idx16 = lax.fori_loop(
            0, _NCHUNK, chunk_body, (zero16, iota16))

        extract_top()
        pltpu.sync_copy(tmp_v, topv.at[row])
        pltpu.sync_copy(tmp_i, topi.at[row])
        s16 = _bsum(s_acc)
        st = jnp.where(iota16 == 0, m16,
                       jnp.where(iota16 == 1, s16, zero16))
        statbuf[...] = st
        pltpu.sync_copy(statbuf, stats.at[row])
        return 0

    lax.fori_loop(0, _ROWS_PER_W, row_body, 0)


_sc_topk = functools.partial(
    pl.kernel,
    out_type=[
        jax.ShapeDtypeStruct((_BATCH, _KPAD), jnp.float32),
        jax.ShapeDtypeStruct((_BATCH, _KPAD), jnp.int32),
        jax.ShapeDtypeStruct((_BATCH, 16), jnp.float32),
    ],
    mesh=plsc.VectorSubcoreMesh(core_axis_name="c", subcore_axis_name="s"),
    scratch_types=[
        pltpu.VMEM((_VOCAB,), jnp.float32),
        pltpu.VMEM((_CAP,), jnp.float32),
        pltpu.VMEM((_CAP,), jnp.int32),
        pltpu.VMEM((_KPAD,), jnp.float32),
        pltpu.VMEM((_KPAD,), jnp.int32),
        pltpu.VMEM((16,), jnp.float32),
        pltpu.VMEM((16,), jnp.float32),
        pltpu.VMEM((32,), jnp.float32),
        pltpu.VMEM((16,), jnp.int32),
        pltpu.SMEM((1,), jnp.int32),
    ],
)(_sc_body)


def _tail_body(topv_ref, topi_ref, stats_ref, g_ref, tok_ref, probs_ref):
    tv = topv_ref[...]
    ti = topi_ref[...]
    st = stats_ref[...]
    g = g_ref[...]
    m = st[:, 0:1]
    s = st[:, 1:2]
    col = lax.broadcasted_iota(jnp.int32, (_BATCH, _KPAD), 1)
    valid = col < _K
    vals = jnp.where(valid, jnp.exp(tv - m) / s, -jnp.inf)
    mx = jnp.max(vals, axis=-1, keepdims=True)
    e = jnp.where(valid, jnp.exp(vals - mx), 0.0)
    sm = e / jnp.sum(e, axis=-1, keepdims=True)
    c = sm
    for sh in (1, 2, 4, 8, 16, 32):
        c = c + jnp.concatenate(
            [jnp.zeros((_BATCH, sh), c.dtype), c[:, :-sh]], axis=-1)
    gt = c > _TOP_P
    any_gt = jnp.any(gt, axis=-1)
    first = jnp.where(any_gt,
                      jnp.min(jnp.where(gt, col, _KPAD), axis=-1), 0)
    first = jnp.minimum(first, _K)
    mask = col < first[:, None]
    num = jnp.exp(c / _TEMP) * mask.astype(jnp.float32)
    den = jnp.sum(num, axis=-1, keepdims=True)
    probs = num / den
    logp = jnp.where(mask, jnp.log(jnp.maximum(probs, 1e-30)), -jnp.inf)
    z = jnp.where(valid, logp + g, -jnp.inf)
    zm = jnp.max(z, axis=-1, keepdims=True)
    j = jnp.min(jnp.where(z == zm, col, _KPAD), axis=-1)
    tok_ref[...] = jnp.sum(jnp.where(col == j[:, None], ti, 0),
                           axis=-1, keepdims=True)
    probs_ref[...] = probs


def kernel(logits, top_k):
    del top_k
    topv, topi, stats = _sc_topk(logits)
    g = jax.random.gumbel(jax.random.key(42), (_BATCH, _K), jnp.float32)
    g = jnp.pad(g, ((0, 0), (0, _KPAD - _K)))
    tok, probs = pl.pallas_call(
        _tail_body,
        out_shape=[
            jax.ShapeDtypeStruct((_BATCH, 1), jnp.int32),
            jax.ShapeDtypeStruct((_BATCH, _KPAD), jnp.float32),
        ],
    )(topv, topi, stats, g)
    return tok[:, 0], probs[:, :_K]


# speedup vs baseline: 1.3583x; 1.0162x over previous
import functools

import jax
import jax.numpy as jnp
from jax import lax
from jax.experimental import pallas as pl
from jax.experimental.pallas import tpu as pltpu
from jax.experimental.pallas import tpu_sc as plsc

_BATCH = 128
_VOCAB = 100000
_K = 50
_KPAD = 64
_TOP_P = 0.9
_TEMP = 0.8

_NWORKERS = 32
_ROWS_PER_W = _BATCH // _NWORKERS
_NVREG = _VOCAB // 16
_CHUNK = 10
_NCHUNK = _NVREG // _CHUNK
_CAP = 512
_BIG = 2 ** 30


def _sc_body(logits, topv, topi, stats, rowbuf, cand_v, cand_i, tmp_v, tmp_i,
             statbuf, thrbuf, digbuf, selbuf, ptr_ref):
    iota16 = lax.iota(jnp.int32, 16)
    neg16 = jnp.full((16,), -jnp.inf, jnp.float32)
    big16 = jnp.full((16,), _BIG, jnp.int32)
    zero16 = jnp.zeros((16,), jnp.float32)
    zeroi16 = jnp.zeros((16,), jnp.int32)
    one16 = jnp.ones((16,), jnp.int32)
    step16 = jnp.full((16,), _CHUNK * 16, jnp.int32)
    thr_lane = jnp.full((16,), (_K - 1) % 16, jnp.int32)
    rot_idx = (iota16 + 15) & 15

    wid = lax.axis_index("c") * 16 + lax.axis_index("s")

    def _g(v, idx):
        return v.at[idx].get(mode="promise_in_bounds")

    def _bmax(v):
        for sh in (8, 4, 2, 1):
            v = jnp.maximum(v, _g(v, iota16 ^ sh))
        return v

    def _bmin(v):
        for sh in (8, 4, 2, 1):
            v = jnp.minimum(v, _g(v, iota16 ^ sh))
        return v

    def _bsum(v):
        for sh in (8, 4, 2, 1):
            v = v + _g(v, iota16 ^ sh)
        return v

    def extract_top():
        def zap(j, _):
            cand_v[pl.ds(j * 16, 16)] = neg16
            return 0
        lax.fori_loop(ptr_ref[0] // 16, _CAP // 16, zap, 0)

        d0, d1 = neg16, neg16
        for j in range(_CAP // 16):
            dv = _bmax(cand_v[pl.ds(j * 16, 16)])
            sel = iota16 == (j % 16)
            if j < 16:
                d0 = jnp.where(sel, dv, d0)
            else:
                d1 = jnp.where(sel, dv, d1)
        digbuf[pl.ds(0, 16)] = d0
        digbuf[pl.ds(16, 16)] = d1

        def ext(t, onehot):
            d0 = digbuf[pl.ds(0, 16)]
            d1 = digbuf[pl.ds(16, 16)]
            m16_ = _bmax(jnp.maximum(d0, d1))
            eq0 = d0 == m16_
            eq1 = d1 == m16_
            j16 = _bmin(jnp.minimum(jnp.where(eq0, iota16, big16),
                                    jnp.where(eq1, iota16 + 16, big16)))
            nt16 = _bsum(jnp.where(eq0, one16, zeroi16) +
                         jnp.where(eq1, one16, zeroi16))
            js = j16[0]

            @pl.when(nt16[0] == 1)
            def _():
                v = cand_v[pl.ds(js * 16, 16)]
                vi = cand_i[pl.ds(js * 16, 16)]
                ti16 = _bmin(jnp.where(v == m16_, vi, big16))
                selbuf[...] = ti16
                v2 = jnp.where(vi == ti16, neg16, v)
                cand_v[pl.ds(js * 16, 16)] = v2
                ndv = _bmax(v2)
                digbuf[pl.ds(0, 16)] = jnp.where(iota16 == j16, ndv, d0)
                digbuf[pl.ds(16, 16)] = jnp.where(iota16 + 16 == j16, ndv, d1)

            @pl.when(nt16[0] > 1)
            def _():
                def msweep(j, acc):
                    v = cand_v[pl.ds(j * 16, 16)]
                    vi = cand_i[pl.ds(j * 16, 16)]
                    return jnp.minimum(acc, jnp.where(v == m16_, vi, big16))
                ti16 = _bmin(lax.fori_loop(0, _CAP // 16, msweep, big16))
                selbuf[...] = ti16

                def killall(j, _):
                    vi = cand_i[pl.ds(j * 16, 16)]
                    cand_v[pl.ds(j * 16, 16)] = jnp.where(
                        vi == ti16, neg16, cand_v[pl.ds(j * 16, 16)])
                    return 0
                lax.fori_loop(0, _CAP // 16, killall, 0)
                nd0, nd1 = neg16, neg16
                for j in range(_CAP // 16):
                    dv = _bmax(cand_v[pl.ds(j * 16, 16)])
                    sel = iota16 == (j % 16)
                    if j < 16:
                        nd0 = jnp.where(sel, dv, nd0)
                    else:
                        nd1 = jnp.where(sel, dv, nd1)
                digbuf[pl.ds(0, 16)] = nd0
                digbuf[pl.ds(16, 16)] = nd1

            ti16 = selbuf[...]
            slot = t // 16 * 16
            lane_sel = onehot > zeroi16
            tmp_v[pl.ds(slot, 16)] = jnp.where(lane_sel, m16_,
                                               tmp_v[pl.ds(slot, 16)])
            tmp_i[pl.ds(slot, 16)] = jnp.where(lane_sel, ti16,
                                               tmp_i[pl.ds(slot, 16)])
            return _g(onehot, rot_idx)
        lax.fori_loop(0, _K, ext, jnp.where(iota16 == 0, one16, zeroi16))

    def row_body(r, _):
        row = wid * _ROWS_PER_W + r
        pltpu.sync_copy(logits.at[row], rowbuf)

        def zap_all(j, _):
            cand_v[pl.ds(j * 16, 16)] = neg16
            return 0
        lax.fori_loop(0, _CAP // 16, zap_all, 0)
        thrbuf[...] = neg16
        ptr_ref[0] = 0

        def max_body(i, accs):
            a0, a1 = accs
            b = i * (_CHUNK * 16)
            for k in range(_CHUNK):
                v = rowbuf[pl.ds(b + k * 16, 16)]
                if k % 2 == 0:
                    a0 = jnp.maximum(a0, v)
                else:
                    a1 = jnp.maximum(a1, v)
            return a0, a1
        a0, a1 = lax.fori_loop(0, _NCHUNK, max_body, (neg16, neg16))
        m16 = _bmax(jnp.maximum(a0, a1))

        def chunk_body(ci_, carry):
            s_acc, idx16 = carry
            base = ci_ * (_CHUNK * 16)
            thr16 = thrbuf[...]
            l0, l1 = zero16, zero16
            o0, o1 = None, None
            for k in range(_CHUNK):
                v = rowbuf[pl.ds(base + k * 16, 16)]
                mk = v > thr16
                if k % 2 == 0:
                    l0 = l0 + jnp.exp(v - m16)
                    o0 = mk if o0 is None else (o0 | mk)
                else:
                    l1 = l1 + jnp.exp(v - m16)
                    o1 = mk if o1 is None else (o1 | mk)
            s_acc = s_acc + (l0 + l1)
            acc_or = o0 | o1
            any16 = _bmax(jnp.where(acc_or, one16, zeroi16))

            @pl.when(any16[0] > 0)
            def _():
                for k in range(_CHUNK):
                    kk16 = jnp.full((16,), k * 16, jnp.int32)
                    v = rowbuf[pl.ds(base + k * 16, 16)]
                    t16 = thrbuf[...]
                    mk = v > t16
                    a16 = _bmax(jnp.where(mk, one16, zeroi16))

                    @pl.when(a16[0] > 0)
                    def _(v=v, mk=mk, kk16=kk16):
                        p = ptr_ref[0]
                        cand_v[pl.ds(p, 16)] = jnp.where(mk, v, neg16)
                        cand_i[pl.ds(p, 16)] = idx16 + kk16
                        ptr_ref[0] = p + 16

                        @pl.when(p + 16 >= _CAP - 16)
                        def _():
                            extract_top()
                            for j in range(3):
                                cand_v[pl.ds(j * 16, 16)] = \
                                    tmp_v[pl.ds(j * 16, 16)]
                                cand_i[pl.ds(j * 16, 16)] = \
                                    tmp_i[pl.ds(j * 16, 16)]
                            cand_v[pl.ds(48, 16)] = jnp.where(
                                iota16 < 2, tmp_v[pl.ds(48, 16)], neg16)
                            cand_i[pl.ds(48, 16)] = tmp_i[pl.ds(48, 16)]

                            def zapr(j, _):
                                cand_v[pl.ds(j * 16, 16)] = neg16
                                return 0
                            lax.fori_loop(_KPAD // 16, _CAP // 16, zapr, 0)
                            thrbuf[...] = _g(tmp_v[pl.ds(48, 16)], thr_lane)
                            ptr_ref[0] = _KPAD

            return s_acc, idx16 + step16

        s_acc, _idx16 = lax.fori_loop(
            0, _NCHUNK, chunk_body, (zero16, iota16))

        extract_top()
        pltpu.sync_copy(tmp_v, topv.at[row])
        pltpu.sync_copy(tmp_i, topi.at[row])
        s16 = _bsum(s_acc)
        st = jnp.where(iota16 == 0, m16,
                       jnp.where(iota16 == 1, s16, zero16))
        statbuf[...] = st
        pltpu.sync_copy(statbuf, stats.at[row])
        return 0

    lax.fori_loop(0, _ROWS_PER_W, row_body, 0)


_sc_topk = functools.partial(
    pl.kernel,
    out_type=[
        jax.ShapeDtypeStruct((_BATCH, _KPAD), jnp.float32),
        jax.ShapeDtypeStruct((_BATCH, _KPAD), jnp.int32),
        jax.ShapeDtypeStruct((_BATCH, 16), jnp.float32),
    ],
    mesh=plsc.VectorSubcoreMesh(core_axis_name="c", subcore_axis_name="s"),
    scratch_types=[
        pltpu.VMEM((_VOCAB,), jnp.float32),
        pltpu.VMEM((_CAP,), jnp.float32),
        pltpu.VMEM((_CAP,), jnp.int32),
        pltpu.VMEM((_KPAD,), jnp.float32),
        pltpu.VMEM((_KPAD,), jnp.int32),
        pltpu.VMEM((16,), jnp.float32),
        pltpu.VMEM((16,), jnp.float32),
        pltpu.VMEM((32,), jnp.float32),
        pltpu.VMEM((16,), jnp.int32),
        pltpu.SMEM((1,), jnp.int32),
    ],
)(_sc_body)


def _tail_body(topv_ref, topi_ref, stats_ref, g_ref, tok_ref, probs_ref):
    tv = topv_ref[...]
    ti = topi_ref[...]
    st = stats_ref[...]
    g = g_ref[...]
    m = st[:, 0:1]
    s = st[:, 1:2]
    col = lax.broadcasted_iota(jnp.int32, (_BATCH, _KPAD), 1)
    valid = col < _K
    vals = jnp.where(valid, jnp.exp(tv - m) / s, -jnp.inf)
    mx = jnp.max(vals, axis=-1, keepdims=True)
    e = jnp.where(valid, jnp.exp(vals - mx), 0.0)
    sm = e / jnp.sum(e, axis=-1, keepdims=True)
    c = sm
    for sh in (1, 2, 4, 8, 16, 32):
        c = c + jnp.concatenate(
            [jnp.zeros((_BATCH, sh), c.dtype), c[:, :-sh]], axis=-1)
    gt = c > _TOP_P
    any_gt = jnp.any(gt, axis=-1)
    first = jnp.where(any_gt,
                      jnp.min(jnp.where(gt, col, _KPAD), axis=-1), 0)
    first = jnp.minimum(first, _K)
    mask = col < first[:, None]
    num = jnp.exp(c / _TEMP) * mask.astype(jnp.float32)
    den = jnp.sum(num, axis=-1, keepdims=True)
    probs = num / den
    logp = jnp.where(mask, jnp.log(jnp.maximum(probs, 1e-30)), -jnp.inf)
    z = jnp.where(valid, logp + g, -jnp.inf)
    zm = jnp.max(z, axis=-1, keepdims=True)
    j = jnp.min(jnp.where(z == zm, col, _KPAD), axis=-1)
    tok_ref[...] = jnp.sum(jnp.where(col == j[:, None], ti, 0),
                           axis=-1, keepdims=True)
    probs_ref[...] = probs


def kernel(logits, top_k):
    del top_k
    topv, topi, stats = _sc_topk(logits)
    g = jax.random.gumbel(jax.random.key(42), (_BATCH, _K), jnp.float32)
    g = jnp.pad(g, ((0, 0), (0, _KPAD - _K)))
    tok, probs = pl.pallas_call(
        _tail_body,
        out_shape=[
            jax.ShapeDtypeStruct((_BATCH, 1), jnp.int32),
            jax.ShapeDtypeStruct((_BATCH, _KPAD), jnp.float32),
        ],
    )(topv, topi, stats, g)
    return tok[:, 0], probs[:, :_K]
